# baseline (speedup 1.0000x reference)
import jax
import jax.numpy as jnp
from jax import lax
from jax.experimental import pallas as pl
from jax.experimental.pallas import tpu as pltpu
from jax.experimental.pallas import tpu_sc as plsc

_ALPHA = 4
_C, _T, _H, _W = 3, 64, 512, 512
_NSLOW = _T // _ALPHA
_SEL = [(63 * j) // 15 for j in range(_NSLOW)]
assert _SEL == [0, 4, 8, 12, 16, 21, 25, 29, 33, 37, 42, 46, 50, 54, 58, 63]
_UNSEL = [21 * (m // 16) + min((m % 16) + 1 + (m % 16) // 3, 20)
          for m in range(_T - _NSLOW)]
assert sorted(_SEL + _UNSEL) == list(range(_T))

_NSLABS = _C * _T
_NSLOWSLABS = _C * _NSLOW
_CHUNK = 64
_CPS = _H // _CHUNK
_NSEL_CH = _NSLOWSLABS * _CPS
_NUNSEL_CH = (_NSLABS - _NSLOWSLABS) * _CPS


def _pack(slabs):
    info = plsc.get_sparse_core_info()
    nw = info.num_cores * info.num_subcores
    assert _NSEL_CH % nw == 0 and _NUNSEL_CH % nw == 0
    na = _NSEL_CH // nw
    nb = _NUNSEL_CH // nw
    n = na + nb
    mesh = plsc.VectorSubcoreMesh(core_axis_name="c", subcore_axis_name="s")

    def body(in_hbm, fast_hbm, slow_hbm,
             buf0, buf1, isem0, isem1, fsem0, fsem1, ssem0, ssem1):
        w = lax.axis_index("c") * info.num_subcores + lax.axis_index("s")
        bufs = (buf0, buf1)
        isems = (isem0, isem1)
        fsems = (fsem0, fsem1)
        ssems = (ssem0, ssem1)

        def unit(b):
            if b < na:
                t = w * na + b
                f = t // _CPS
                r = (t % _CPS) * _CHUNK
                ch = f // _NSLOW
                j = f % _NSLOW
                src = ch * _T + (63 * j) // 15
                return src, r, f
            t = w * nb + (b - na)
            k = t // _CPS
            r = (t % _CPS) * _CHUNK
            ch = k // (_T - _NSLOW)
            m = k % (_T - _NSLOW)
            p = m % 16
            o = jnp.minimum(p + 1 + p // 3, 20)
            src = ch * _T + 21 * (m // 16) + o
            return src, r, None

        def start_in(b):
            src, r, _ = unit(b)
            c = pltpu.make_async_copy(
                in_hbm.at[src, pl.ds(r, _CHUNK), :],
                bufs[b % 2], isems[b % 2])
            c.start()
            return c

        def start_outs(b):
            src, r, f = unit(b)
            cs = []
            c = pltpu.make_async_copy(
                bufs[b % 2], fast_hbm.at[src, pl.ds(r, _CHUNK), :],
                fsems[b % 2])
            c.start()
            cs.append(c)
            if f is not None:
                c2 = pltpu.make_async_copy(
                    bufs[b % 2], slow_hbm.at[f, pl.ds(r, _CHUNK), :],
                    ssems[b % 2])
                c2.start()
                cs.append(c2)
            return cs

        cin = [None] * n
        cout = [None] * n
        cin[0] = start_in(0)
        for b in range(n):
            if b + 1 < n:
                if b >= 1:
                    for c in cout[b - 1]:
                        c.wait()
                cin[b + 1] = start_in(b + 1)
            cin[b].wait()
            cout[b] = start_outs(b)
        for c in cout[n - 2] + cout[n - 1]:
            c.wait()

    run = pl.kernel(
        body,
        out_type=(
            jax.ShapeDtypeStruct((_NSLABS, _H, _W), jnp.float32),
            jax.ShapeDtypeStruct((_NSLOWSLABS, _H, _W), jnp.float32),
        ),
        mesh=mesh,
        scratch_types=[
            pltpu.VMEM((_CHUNK, _W), jnp.float32),
            pltpu.VMEM((_CHUNK, _W), jnp.float32),
            pltpu.SemaphoreType.DMA,
            pltpu.SemaphoreType.DMA,
            pltpu.SemaphoreType.DMA,
            pltpu.SemaphoreType.DMA,
            pltpu.SemaphoreType.DMA,
            pltpu.SemaphoreType.DMA,
        ],
    )
    return run(slabs)


def kernel(frames):
    slabs = frames.reshape(_NSLABS, _H, _W)
    fast, slow = _pack(slabs)
    return (slow.reshape(_C, _NSLOW, _H, _W), fast.reshape(_C, _T, _H, _W))


# speedup vs baseline: 1.0079x; 1.0079x over previous
import jax
import jax.numpy as jnp
from jax import lax
from jax.experimental import pallas as pl
from jax.experimental.pallas import tpu as pltpu
from jax.experimental.pallas import tpu_sc as plsc

_ALPHA = 4
_C, _T, _H, _W = 3, 64, 512, 512
_NSLOW = _T // _ALPHA
_SEL = [(63 * j) // 15 for j in range(_NSLOW)]
assert _SEL == [0, 4, 8, 12, 16, 21, 25, 29, 33, 37, 42, 46, 50, 54, 58, 63]
_UNSEL = [21 * (m // 16) + min((m % 16) + 1 + (m % 16) // 3, 20)
          for m in range(_T - _NSLOW)]
assert sorted(_SEL + _UNSEL) == list(range(_T))

_NSLABS = _C * _T
_NSLOWSLABS = _C * _NSLOW
_CHUNK = 64
_CPS = _H // _CHUNK
_NSEL_CH = _NSLOWSLABS * _CPS
_NUNSEL_CH = (_NSLABS - _NSLOWSLABS) * _CPS


def _pack(slabs):
    info = plsc.get_sparse_core_info()
    nw = info.num_cores * info.num_subcores
    assert _NSEL_CH % nw == 0 and _NUNSEL_CH % nw == 0
    na = _NSEL_CH // nw
    nb = _NUNSEL_CH // nw
    n = na + nb
    mesh = plsc.VectorSubcoreMesh(core_axis_name="c", subcore_axis_name="s")

    nbuf = 3

    def body(in_hbm, fast_hbm, slow_hbm, *rest):
        w = lax.axis_index("c") * info.num_subcores + lax.axis_index("s")
        bufs = rest[0:nbuf]
        isems = rest[nbuf:2 * nbuf]
        fsems = rest[2 * nbuf:3 * nbuf]
        ssems = rest[3 * nbuf:4 * nbuf]

        def unit(b):
            if b < na:
                t = w * na + b
                f = t // _CPS
                r = (t % _CPS) * _CHUNK
                ch = f // _NSLOW
                j = f % _NSLOW
                src = ch * _T + (63 * j) // 15
                return src, r, f
            t = w * nb + (b - na)
            k = t // _CPS
            r = (t % _CPS) * _CHUNK
            ch = k // (_T - _NSLOW)
            m = k % (_T - _NSLOW)
            p = m % 16
            o = jnp.minimum(p + 1 + p // 3, 20)
            src = ch * _T + 21 * (m // 16) + o
            return src, r, None

        def start_in(b):
            src, r, _ = unit(b)
            c = pltpu.make_async_copy(
                in_hbm.at[src, pl.ds(r, _CHUNK), :],
                bufs[b % nbuf], isems[b % nbuf])
            c.start()
            return c

        def start_outs(b):
            src, r, f = unit(b)
            cs = []
            c = pltpu.make_async_copy(
                bufs[b % nbuf], fast_hbm.at[src, pl.ds(r, _CHUNK), :],
                fsems[b % nbuf])
            c.start()
            cs.append(c)
            if f is not None:
                c2 = pltpu.make_async_copy(
                    bufs[b % nbuf], slow_hbm.at[f, pl.ds(r, _CHUNK), :],
                    ssems[b % nbuf])
                c2.start()
                cs.append(c2)
            return cs

        cin = [None] * n
        cout = [None] * n
        for g in range(nbuf - 1):
            cin[g] = start_in(g)
        for b in range(n):
            nxt = b + nbuf - 1
            if nxt < n:
                if b >= 1:
                    for c in cout[b - 1]:
                        c.wait()
                cin[nxt] = start_in(nxt)
            cin[b].wait()
            cout[b] = start_outs(b)
        for b in range(max(0, n - nbuf + 1), n):
            for c in cout[b]:
                c.wait()

    run = pl.kernel(
        body,
        out_type=(
            jax.ShapeDtypeStruct((_NSLABS, _H, _W), jnp.float32),
            jax.ShapeDtypeStruct((_NSLOWSLABS, _H, _W), jnp.float32),
        ),
        mesh=mesh,
        scratch_types=(
            [pltpu.VMEM((_CHUNK, _W), jnp.float32)] * 3
            + [pltpu.SemaphoreType.DMA] * 9),
    )
    return run(slabs)


def kernel(frames):
    slabs = frames.reshape(_NSLABS, _H, _W)
    fast, slow = _pack(slabs)
    return (slow.reshape(_C, _NSLOW, _H, _W), fast.reshape(_C, _T, _H, _W))
